# baseline (device time: 262299 ns/iter reference)
import jax
import jax.numpy as jnp
from jax import lax
from jax.experimental import pallas as pl
from jax.experimental.pallas import tpu as pltpu

T = 2048
D = 4096
V_SHARD = 8192
VB = 1024
NB = V_SHARD // VB


def kernel(x, W, labels):
    x = x.astype(jnp.bfloat16)
    W = W.astype(jnp.bfloat16)
    labels2d = labels.reshape(T, 1)

    def body(x_ref, w_ref, lab_ref, out_ref,
             s_ref, lg_ref,
             comm_send, comm_recv, send_sem, recv_sem):
        i = pl.program_id(0)
        my_x = lax.axis_index("x")
        my_y = lax.axis_index("y")

        @pl.when(i == 0)
        def _init():
            s_ref[...] = jnp.zeros((T, 1), jnp.float32)
            lg_ref[...] = jnp.zeros((T, 1), jnp.float32)

        logits = jnp.dot(x_ref[...], w_ref[...],
                         preferred_element_type=jnp.float32)

        s_ref[...] = s_ref[...] + jnp.sum(jnp.exp(logits), axis=1,
                                          keepdims=True)

        v0 = my_x * V_SHARD + i * VB
        col = lax.broadcasted_iota(jnp.int32, (T, VB), 1)
        hit = col == (lab_ref[...] - v0)
        lg_ref[...] = lg_ref[...] + jnp.sum(
            jnp.where(hit, logits, 0.0), axis=1, keepdims=True)

        @pl.when(i == NB - 1)
        def _exchange():
            comm_send[:, 0:1] = s_ref[...]
            comm_send[:, 1:2] = lg_ref[...]

            partner = (1 - my_x, my_y)
            barrier_sem = pltpu.get_barrier_semaphore()
            pl.semaphore_signal(barrier_sem, inc=1, device_id=partner,
                                device_id_type=pl.DeviceIdType.MESH)
            pl.semaphore_wait(barrier_sem, 1)

            rdma = pltpu.make_async_remote_copy(
                src_ref=comm_send, dst_ref=comm_recv,
                send_sem=send_sem, recv_sem=recv_sem,
                device_id=partner, device_id_type=pl.DeviceIdType.MESH)
            rdma.start()
            rdma.wait()

            s_tot = s_ref[...] + comm_recv[:, 0:1]
            lg_tot = lg_ref[...] + comm_recv[:, 1:2]
            out_ref[...] = jnp.log(s_tot) - lg_tot

    out = pl.pallas_call(
        body,
        grid=(NB,),
        in_specs=[
            pl.BlockSpec((T, D), lambda i: (0, 0)),
            pl.BlockSpec((D, VB), lambda i: (0, i)),
            pl.BlockSpec((T, 1), lambda i: (0, 0)),
        ],
        out_specs=pl.BlockSpec((T, 1), lambda i: (0, 0)),
        out_shape=jax.ShapeDtypeStruct((T, 1), jnp.float32),
        scratch_shapes=[
            pltpu.VMEM((T, 1), jnp.float32),
            pltpu.VMEM((T, 1), jnp.float32),
            pltpu.VMEM((T, 2), jnp.float32),
            pltpu.VMEM((T, 2), jnp.float32),
            pltpu.SemaphoreType.DMA,
            pltpu.SemaphoreType.DMA,
        ],
        compiler_params=pltpu.CompilerParams(
            dimension_semantics=("arbitrary",),
            collective_id=0,
            vmem_limit_bytes=96 * 1024 * 1024,
        ),
    )(x, W, labels2d)
    return out[:, 0]
